# baseline (speedup 1.0000x reference)
import functools

import jax
import jax.numpy as jnp
from jax.experimental import pallas as pl
from jax.experimental.pallas import tpu as pltpu

_C_HI, _C_LO = 14, 4
_LANE = 128


def _reduce_body(x_ref, msum_ref, csum_ref, *, nblk, layers_per_block,
                 rows_per_layer, c_hi, c_lo):
    hb = pl.program_id(0)
    b = pl.program_id(1)
    blk = hb * nblk + b

    @pl.when(b == 0)
    def _init():
        msum_ref[...] = jnp.zeros_like(msum_ref)
        csum_ref[...] = jnp.zeros_like(csum_ref)

    x = x_ref[...]
    k, rpl = layers_per_block, rows_per_layer
    vp = x.shape[-1]
    parts = x.reshape(k, rpl // 8, 8, vp).astype(jnp.float32).sum(axis=1)
    msum_ref[0] += parts.sum(axis=0)

    @pl.when(blk == c_hi // k)
    def _hi():
        csum_ref[0] += parts[c_hi % k]

    @pl.when(blk == c_lo // k)
    def _lo():
        csum_ref[0] -= parts[c_lo % k]


def _attn_reduce(image_attn, c_hi, c_lo, layers_per_block=1):
    L, H, Q, Vp = image_attn.shape
    assert Vp % _LANE == 0
    assert L % 2 == 0, "megacore split over layer halves needs even L"
    assert (L // 2) % layers_per_block == 0

    rows_per_layer = H * Q
    rows = L * rows_per_layer
    block_rows = layers_per_block * rows_per_layer
    nblk = (rows // 2) // block_rows

    flat = image_attn.reshape(rows, Vp)

    body = functools.partial(
        _reduce_body, nblk=nblk, layers_per_block=layers_per_block,
        rows_per_layer=rows_per_layer, c_hi=c_hi, c_lo=c_lo)

    msum, csum = pl.pallas_call(
        body,
        out_shape=(
            jax.ShapeDtypeStruct((2, 8, Vp), jnp.float32),
            jax.ShapeDtypeStruct((2, 8, Vp), jnp.float32),
        ),
        grid=(2, nblk),
        in_specs=[pl.BlockSpec((block_rows, Vp),
                               lambda hb, b: (hb * nblk + b, 0))],
        out_specs=(
            pl.BlockSpec((1, 8, Vp), lambda hb, b: (hb, 0, 0)),
            pl.BlockSpec((1, 8, Vp), lambda hb, b: (hb, 0, 0)),
        ),
        compiler_params=pltpu.CompilerParams(
            dimension_semantics=("parallel", "arbitrary")),
    )(flat)

    mean_attn = jnp.sum(msum, axis=(0, 1)) / float(rows)
    contr = jnp.sum(csum, axis=(0, 1)) / float(rows_per_layer)
    return mean_attn, jnp.maximum(contr, 0.0)


def kernel(image_attn):
    return _attn_reduce(image_attn, _C_HI, _C_LO, layers_per_block=2)
